# baseline (speedup 1.0000x reference)
import functools

import jax
import jax.numpy as jnp
from jax import lax
from jax.experimental import pallas as pl
from jax.experimental.pallas import tpu as pltpu
from jax.experimental.pallas import tpu_sc as plsc

DIM = 32
SCALE = 10.0

B = 16384 * 50
NC = 2
NS = 16
NW = NC * NS
B_PER_W = B // NW
CHUNK = 1024
N_CHUNKS = B_PER_W // CHUNK


def _sc_body(idx_hbm, w_hbm, out_hbm, idx_v, rows_v, sem):
    wid = lax.axis_index("s") * NC + lax.axis_index("c")
    base = wid * B_PER_W

    def chunk(g, carry):
        off = base + g * CHUNK
        pltpu.sync_copy(idx_hbm.at[pl.ds(off, CHUNK)], idx_v)
        pltpu.async_copy(w_hbm.at[idx_v], rows_v, sem).wait()

        def scale(i, c):
            rows_v[i, pl.ds(0, 16)] = rows_v[i, pl.ds(0, 16)] * SCALE
            rows_v[i, pl.ds(16, 16)] = rows_v[i, pl.ds(16, 16)] * SCALE
            return c

        lax.fori_loop(0, CHUNK, scale, 0)
        pltpu.sync_copy(rows_v, out_hbm.at[pl.ds(off, CHUNK)])
        return carry

    lax.fori_loop(0, N_CHUNKS, chunk, 0)


def kernel(x, weight):
    xf = x.reshape(-1).astype(jnp.int32)
    mesh = plsc.VectorSubcoreMesh(core_axis_name="c", subcore_axis_name="s")
    out = pl.kernel(
        _sc_body,
        out_type=jax.ShapeDtypeStruct((B, DIM), jnp.float32),
        mesh=mesh,
        scratch_types=[
            pltpu.VMEM((CHUNK,), jnp.int32),
            pltpu.VMEM((CHUNK, DIM), jnp.float32),
            pltpu.SemaphoreType.DMA,
        ],
        compiler_params=pltpu.CompilerParams(use_tc_tiling_on_sc=False),
    )(xf, weight)
    return out.reshape(x.shape + (DIM,))


# speedup vs baseline: 1.0828x; 1.0828x over previous
import jax
import jax.numpy as jnp
from jax import lax
from jax.experimental import pallas as pl
from jax.experimental.pallas import tpu as pltpu
from jax.experimental.pallas import tpu_sc as plsc

DIM = 32
SCALE = 10.0

B = 16384 * 50
NC = 2
NS = 16
NW = NC * NS
B_PER_W = B // NW
CHUNK = 1600
N_CHUNKS = B_PER_W // CHUNK
UNROLL = 4


def _scale_rows(rows):
    def body(i, c):
        r = i * UNROLL
        for u in range(UNROLL):
            rows[r + u, pl.ds(0, 16)] = rows[r + u, pl.ds(0, 16)] * SCALE
            rows[r + u, pl.ds(16, 16)] = rows[r + u, pl.ds(16, 16)] * SCALE
        return c

    lax.fori_loop(0, CHUNK // UNROLL, body, 0)


def _sc_body(idx_hbm, w_hbm, out_hbm,
             idx0, idx1, rows0, rows1, gsem0, gsem1, osem0, osem1):
    wid = lax.axis_index("s") * NC + lax.axis_index("c")
    base = wid * B_PER_W

    idx_v = (idx0, idx1)
    rows_v = (rows0, rows1)
    gsem = (gsem0, gsem1)
    osem = (osem0, osem1)

    gathers = [None, None]
    writes = [None, None]

    pltpu.sync_copy(idx_hbm.at[pl.ds(base, CHUNK)], idx_v[0])
    gathers[0] = pltpu.async_copy(w_hbm.at[idx_v[0]], rows_v[0], gsem[0])

    for g in range(N_CHUNKS):
        b = g % 2
        nb = (g + 1) % 2
        if g + 1 < N_CHUNKS:
            if writes[nb] is not None:
                writes[nb].wait()
            off = base + (g + 1) * CHUNK
            pltpu.sync_copy(idx_hbm.at[pl.ds(off, CHUNK)], idx_v[nb])
            gathers[nb] = pltpu.async_copy(
                w_hbm.at[idx_v[nb]], rows_v[nb], gsem[nb])
        gathers[b].wait()
        _scale_rows(rows_v[b])
        writes[b] = pltpu.async_copy(
            rows_v[b], out_hbm.at[pl.ds(base + g * CHUNK, CHUNK)], osem[b])

    writes[0].wait()
    writes[1].wait()


def kernel(x, weight):
    xf = x.reshape(-1).astype(jnp.int32)
    mesh = plsc.VectorSubcoreMesh(core_axis_name="c", subcore_axis_name="s")
    out = pl.kernel(
        _sc_body,
        out_type=jax.ShapeDtypeStruct((B, DIM), jnp.float32),
        mesh=mesh,
        scratch_types=[
            pltpu.VMEM((CHUNK,), jnp.int32),
            pltpu.VMEM((CHUNK,), jnp.int32),
            pltpu.VMEM((CHUNK, DIM), jnp.float32),
            pltpu.VMEM((CHUNK, DIM), jnp.float32),
            pltpu.SemaphoreType.DMA,
            pltpu.SemaphoreType.DMA,
            pltpu.SemaphoreType.DMA,
            pltpu.SemaphoreType.DMA,
        ],
        compiler_params=pltpu.CompilerParams(use_tc_tiling_on_sc=False),
    )(xf, weight)
    return out.reshape(x.shape + (DIM,))


# speedup vs baseline: 1.3843x; 1.2784x over previous
import jax
import jax.numpy as jnp
from jax import lax
from jax.experimental import pallas as pl
from jax.experimental.pallas import tpu as pltpu
from jax.experimental.pallas import tpu_sc as plsc

ROWS = 16384
COLS = 50
DIM = 32
SCALE = 10.0

NC = 2
NS = 16
NW = NC * NS
R_PER_W = ROWS // NW
CHI = 16
N_CHUNKS = R_PER_W // CHI


def _sc_body(x_hbm, w_hbm, out_hbm,
             idx0, idx1, rows0, rows1, gsem0, gsem1, osem0, osem1):
    wid = lax.axis_index("s") * NC + lax.axis_index("c")
    base = wid * R_PER_W

    idx_v = (idx0, idx1)
    rows_v = (rows0, rows1)
    gsem = (gsem0, gsem1)
    osem = (osem0, osem1)

    gathers = [None, None]
    writes = [None, None]

    def stage(g, b):
        i0 = base + g * CHI
        pltpu.sync_copy(x_hbm.at[pl.ds(i0, CHI)], idx_v[b])
        gathers[b] = [
            pltpu.async_copy(
                w_hbm.at[idx_v[b].at[r]], rows_v[b].at[r], gsem[b])
            for r in range(CHI)
        ]

    def scale(b):
        def body_r(r, c):
            def body_j(j, c2):
                rows_v[b][r, j, pl.ds(0, 16)] = (
                    rows_v[b][r, j, pl.ds(0, 16)] * SCALE)
                rows_v[b][r, j, pl.ds(16, 16)] = (
                    rows_v[b][r, j, pl.ds(16, 16)] * SCALE)
                return c2
            return lax.fori_loop(0, COLS, body_j, c)
        lax.fori_loop(0, CHI, body_r, 0)

    stage(0, 0)
    for g in range(N_CHUNKS):
        b = g % 2
        nb = (g + 1) % 2
        if g + 1 < N_CHUNKS:
            if writes[nb] is not None:
                writes[nb].wait()
            stage(g + 1, nb)
        for cp in gathers[b]:
            cp.wait()
        scale(b)
        writes[b] = pltpu.async_copy(
            rows_v[b], out_hbm.at[pl.ds(base + g * CHI, CHI)], osem[b])

    writes[0].wait()
    writes[1].wait()


def kernel(x, weight):
    mesh = plsc.VectorSubcoreMesh(core_axis_name="c", subcore_axis_name="s")
    out = pl.kernel(
        _sc_body,
        out_type=jax.ShapeDtypeStruct((ROWS, COLS, DIM), jnp.float32),
        mesh=mesh,
        scratch_types=[
            pltpu.VMEM((CHI, COLS), jnp.int32),
            pltpu.VMEM((CHI, COLS), jnp.int32),
            pltpu.VMEM((CHI, COLS, DIM), jnp.float32),
            pltpu.VMEM((CHI, COLS, DIM), jnp.float32),
            pltpu.SemaphoreType.DMA,
            pltpu.SemaphoreType.DMA,
            pltpu.SemaphoreType.DMA,
            pltpu.SemaphoreType.DMA,
        ],
        compiler_params=pltpu.CompilerParams(use_tc_tiling_on_sc=False),
    )(x, weight)
    return out
